# Initial kernel scaffold; baseline (speedup 1.0000x reference)
import jax
import jax.numpy as jnp
from jax.experimental import pallas as pl


def kernel(x, M1, M2, G, idx1, idx2, dist1, dist2):
    raise NotImplementedError("write your pallas kernel here")


# speedup vs baseline: 4.0944x; 4.0944x over previous
import functools

import jax
import jax.numpy as jnp
from jax import lax
from jax.experimental import pallas as pl
from jax.experimental.pallas import tpu as pltpu
from jax.experimental.pallas import tpu_sc as plsc

S1, S2 = 8192, 4096
R = 16
K = 8

_info = plsc.get_sparse_core_info()
_NC, _NS = _info.num_cores, _info.num_subcores
_NW = _NC * _NS

_N1 = S1 * K
_N2 = S2 * K
_C1 = _N1 // _NW
_C2 = _N2 // _NW


def _sc_gather_body(m1_hbm, i1_hbm, m2_hbm, i2_hbm, o1_hbm, o2_hbm,
                    i1_v, r1_v, i2_v, r2_v, sem):
    wid = lax.axis_index("s") * _NC + lax.axis_index("c")
    b1 = wid * _C1
    pltpu.sync_copy(i1_hbm.at[pl.ds(b1, _C1)], i1_v)
    pltpu.async_copy(m1_hbm.at[i1_v], r1_v, sem).wait()
    pltpu.sync_copy(r1_v, o1_hbm.at[pl.ds(b1, _C1)])
    b2 = wid * _C2
    pltpu.sync_copy(i2_hbm.at[pl.ds(b2, _C2)], i2_v)
    pltpu.async_copy(m2_hbm.at[i2_v], r2_v, sem).wait()
    pltpu.sync_copy(r2_v, o2_hbm.at[pl.ds(b2, _C2)])


_sc_gather = functools.partial(
    pl.kernel,
    out_type=(
        jax.ShapeDtypeStruct((_N1, R), jnp.float32),
        jax.ShapeDtypeStruct((_N2, R), jnp.float32),
    ),
    mesh=plsc.VectorSubcoreMesh(core_axis_name="c", subcore_axis_name="s"),
    scratch_types=[
        pltpu.VMEM((_C1,), jnp.int32),
        pltpu.VMEM((_C1, R), jnp.float32),
        pltpu.VMEM((_C2,), jnp.int32),
        pltpu.VMEM((_C2, R), jnp.float32),
        pltpu.SemaphoreType.DMA,
    ],
    compiler_params=pltpu.CompilerParams(use_tc_tiling_on_sc=False),
)(_sc_gather_body)


_BI = 256


def _tc_body(w1_ref, r1_ref, w2_ref, r2_ref, g_ref, out_ref, mx2t_ref):
    i = pl.program_id(0)

    @pl.when(i == 0)
    def _():
        acc = w2_ref[:, 0:1] * r2_ref[:, 0:R]
        for k in range(1, K):
            acc = acc + w2_ref[:, k:k + 1] * r2_ref[:, k * R:(k + 1) * R]
        mx2t_ref[...] = acc.T

    mx1 = w1_ref[:, 0:1] * r1_ref[:, 0:R]
    for k in range(1, K):
        mx1 = mx1 + w1_ref[:, k:k + 1] * r1_ref[:, k * R:(k + 1) * R]
    a = jnp.dot(mx1, g_ref[...], preferred_element_type=jnp.float32)
    out_ref[...] = jnp.dot(a, mx2t_ref[...],
                           preferred_element_type=jnp.float32)


def kernel(x, M1, M2, G, idx1, idx2, dist1, dist2):
    del x
    r1, r2 = _sc_gather(M1, idx1.reshape(-1), M2, idx2.reshape(-1))
    r1f = r1.reshape(S1, K * R)
    r2f = r2.reshape(S2, K * R)

    grid = (S1 // _BI,)
    out = pl.pallas_call(
        _tc_body,
        grid=grid,
        in_specs=[
            pl.BlockSpec((_BI, K), lambda i: (i, 0)),
            pl.BlockSpec((_BI, K * R), lambda i: (i, 0)),
            pl.BlockSpec((S2, K), lambda i: (0, 0)),
            pl.BlockSpec((S2, K * R), lambda i: (0, 0)),
            pl.BlockSpec((R, R), lambda i: (0, 0)),
        ],
        out_specs=pl.BlockSpec((_BI, S2), lambda i: (i, 0)),
        out_shape=jax.ShapeDtypeStruct((S1, S2), jnp.float32),
        scratch_shapes=[pltpu.VMEM((R, S2), jnp.float32)],
    )(dist1, r1f, dist2, r2f, G)
    return out


# speedup vs baseline: 4.4038x; 1.0756x over previous
import functools

import jax
import jax.numpy as jnp
from jax import lax
from jax.experimental import pallas as pl
from jax.experimental.pallas import tpu as pltpu
from jax.experimental.pallas import tpu_sc as plsc

S1, S2 = 8192, 4096
R = 16
K = 8

_info = plsc.get_sparse_core_info()
_NC, _NS = _info.num_cores, _info.num_subcores
_NW = _NC * _NS

_N1 = S1 * K
_N2 = S2 * K
_C1 = _N1 // _NW
_C2 = _N2 // _NW


def _sc_gather_body(m1_hbm, i1_hbm, m2_hbm, i2_hbm, o1_hbm, o2_hbm,
                    i1_v, r1_v, i2_v, r2_v, sem):
    wid = lax.axis_index("s") * _NC + lax.axis_index("c")
    b1 = wid * _C1
    pltpu.sync_copy(i1_hbm.at[pl.ds(b1, _C1)], i1_v)
    pltpu.async_copy(m1_hbm.at[i1_v], r1_v, sem).wait()
    pltpu.sync_copy(r1_v, o1_hbm.at[pl.ds(b1, _C1)])
    b2 = wid * _C2
    pltpu.sync_copy(i2_hbm.at[pl.ds(b2, _C2)], i2_v)
    pltpu.async_copy(m2_hbm.at[i2_v], r2_v, sem).wait()
    pltpu.sync_copy(r2_v, o2_hbm.at[pl.ds(b2, _C2)])


_sc_gather = functools.partial(
    pl.kernel,
    out_type=(
        jax.ShapeDtypeStruct((_N1, R), jnp.float32),
        jax.ShapeDtypeStruct((_N2, R), jnp.float32),
    ),
    mesh=plsc.VectorSubcoreMesh(core_axis_name="c", subcore_axis_name="s"),
    scratch_types=[
        pltpu.VMEM((_C1,), jnp.int32),
        pltpu.VMEM((_C1, R), jnp.float32),
        pltpu.VMEM((_C2,), jnp.int32),
        pltpu.VMEM((_C2, R), jnp.float32),
        pltpu.SemaphoreType.DMA,
    ],
    compiler_params=pltpu.CompilerParams(use_tc_tiling_on_sc=False),
)(_sc_gather_body)


_BI = 512


def _prep_body(w2_ref, r2_ref, g_ref, b_ref):
    acc = w2_ref[:, 0:1] * r2_ref[:, 0:R]
    for k in range(1, K):
        acc = acc + w2_ref[:, k:k + 1] * r2_ref[:, k * R:(k + 1) * R]
    b_ref[...] = lax.dot_general(
        g_ref[...], acc, (((1,), (1,)), ((), ())),
        preferred_element_type=jnp.float32)


def _main_body(w1_ref, r1_ref, b_ref, out_ref):
    mx1 = w1_ref[:, 0:1] * r1_ref[:, 0:R]
    for k in range(1, K):
        mx1 = mx1 + w1_ref[:, k:k + 1] * r1_ref[:, k * R:(k + 1) * R]
    out_ref[...] = jnp.dot(mx1, b_ref[...],
                           preferred_element_type=jnp.float32)


def kernel(x, M1, M2, G, idx1, idx2, dist1, dist2):
    del x
    r1, r2 = _sc_gather(M1, idx1.reshape(-1), M2, idx2.reshape(-1))
    r1f = r1.reshape(S1, K * R)
    r2f = r2.reshape(S2, K * R)

    b = pl.pallas_call(
        _prep_body,
        out_shape=jax.ShapeDtypeStruct((R, S2), jnp.float32),
    )(dist2, r2f, G)

    out = pl.pallas_call(
        _main_body,
        grid=(S1 // _BI,),
        in_specs=[
            pl.BlockSpec((_BI, K), lambda i: (i, 0)),
            pl.BlockSpec((_BI, K * R), lambda i: (i, 0)),
            pl.BlockSpec((R, S2), lambda i: (0, 0)),
        ],
        out_specs=pl.BlockSpec((_BI, S2), lambda i: (i, 0)),
        out_shape=jax.ShapeDtypeStruct((S1, S2), jnp.float32),
        compiler_params=pltpu.CompilerParams(
            dimension_semantics=("parallel",)),
    )(dist1, r1f, b)
    return out


# speedup vs baseline: 4.5885x; 1.0419x over previous
import functools

import jax
import jax.numpy as jnp
from jax import lax
from jax.experimental import pallas as pl
from jax.experimental.pallas import tpu as pltpu
from jax.experimental.pallas import tpu_sc as plsc

S1, S2 = 8192, 4096
R = 16
K = 8

_info = plsc.get_sparse_core_info()
_NC, _NS = _info.num_cores, _info.num_subcores
_NW = _NC * _NS

_N1 = S1 * K
_N2 = S2 * K
_C1 = _N1 // _NW
_C2 = _N2 // _NW


def _make_sc_gather(n_rows, chunk):
    def body(m_hbm, i_hbm, o_hbm, i_v, r_v, sem):
        wid = lax.axis_index("s") * _NC + lax.axis_index("c")
        b = wid * chunk
        pltpu.sync_copy(i_hbm.at[pl.ds(b, chunk)], i_v)
        pltpu.async_copy(m_hbm.at[i_v], r_v, sem).wait()
        pltpu.sync_copy(r_v, o_hbm.at[pl.ds(b, chunk)])

    return functools.partial(
        pl.kernel,
        out_type=jax.ShapeDtypeStruct((n_rows, R), jnp.float32),
        mesh=plsc.VectorSubcoreMesh(core_axis_name="c",
                                    subcore_axis_name="s"),
        scratch_types=[
            pltpu.VMEM((chunk,), jnp.int32),
            pltpu.VMEM((chunk, R), jnp.float32),
            pltpu.SemaphoreType.DMA,
        ],
        compiler_params=pltpu.CompilerParams(use_tc_tiling_on_sc=False),
    )(body)


_sc_gather1 = _make_sc_gather(_N1, _C1)
_sc_gather2 = _make_sc_gather(_N2, _C2)


_BI = 1024


def _prep_body(w2_ref, r2_ref, g_ref, b_ref):
    acc = w2_ref[:, 0:1] * r2_ref[:, 0:R]
    for k in range(1, K):
        acc = acc + w2_ref[:, k:k + 1] * r2_ref[:, k * R:(k + 1) * R]
    b_ref[...] = lax.dot_general(
        g_ref[...], acc, (((1,), (1,)), ((), ())),
        preferred_element_type=jnp.float32)


def _main_body(w1_ref, r1_ref, b_ref, out_ref):
    mx1 = w1_ref[:, 0:1] * r1_ref[:, 0:R]
    for k in range(1, K):
        mx1 = mx1 + w1_ref[:, k:k + 1] * r1_ref[:, k * R:(k + 1) * R]
    out_ref[...] = jnp.dot(mx1, b_ref[...],
                           preferred_element_type=jnp.float32)


def kernel(x, M1, M2, G, idx1, idx2, dist1, dist2):
    del x
    r2 = _sc_gather2(M2, idx2.reshape(-1))
    r1 = _sc_gather1(M1, idx1.reshape(-1))
    r1f = r1.reshape(S1, K * R)
    r2f = r2.reshape(S2, K * R)

    b = pl.pallas_call(
        _prep_body,
        out_shape=jax.ShapeDtypeStruct((R, S2), jnp.float32),
    )(dist2, r2f, G)

    out = pl.pallas_call(
        _main_body,
        grid=(S1 // _BI,),
        in_specs=[
            pl.BlockSpec((_BI, K), lambda i: (i, 0)),
            pl.BlockSpec((_BI, K * R), lambda i: (i, 0)),
            pl.BlockSpec((R, S2), lambda i: (0, 0)),
        ],
        out_specs=pl.BlockSpec((_BI, S2), lambda i: (i, 0)),
        out_shape=jax.ShapeDtypeStruct((S1, S2), jnp.float32),
        compiler_params=pltpu.CompilerParams(
            dimension_semantics=("parallel",)),
    )(dist1, r1f, b)
    return out


# speedup vs baseline: 5.0433x; 1.0991x over previous
import functools

import jax
import jax.numpy as jnp
from jax import lax
from jax.experimental import pallas as pl
from jax.experimental.pallas import tpu as pltpu
from jax.experimental.pallas import tpu_sc as plsc

S1, S2 = 8192, 4096
R = 16
K = 8

_info = plsc.get_sparse_core_info()
_NC, _NS = _info.num_cores, _info.num_subcores
_NW = _NC * _NS

_N1 = S1 * K
_N2 = S2 * K
_C1 = _N1 // _NW
_C2 = _N2 // _NW


def _make_sc_gather(n_table, n_rows, chunk):
    def body(m_hbm, i_hbm, o_hbm, m_sh, i_v, r_v, sem):
        sid = lax.axis_index("s")
        wid = sid * _NC + lax.axis_index("c")
        b = wid * chunk

        @pl.when(sid == 0)
        def _():
            pltpu.sync_copy(m_hbm, m_sh)

        pltpu.sync_copy(i_hbm.at[pl.ds(b, chunk)], i_v)
        plsc.subcore_barrier()
        pltpu.async_copy(m_sh.at[i_v], r_v, sem).wait()
        pltpu.sync_copy(r_v, o_hbm.at[pl.ds(b, chunk)])

    return functools.partial(
        pl.kernel,
        out_type=jax.ShapeDtypeStruct((n_rows, R), jnp.float32),
        mesh=plsc.VectorSubcoreMesh(core_axis_name="c",
                                    subcore_axis_name="s"),
        scratch_types=[
            pltpu.VMEM_SHARED((n_table, R), jnp.float32),
            pltpu.VMEM((chunk,), jnp.int32),
            pltpu.VMEM((chunk, R), jnp.float32),
            pltpu.SemaphoreType.DMA,
        ],
        compiler_params=pltpu.CompilerParams(use_tc_tiling_on_sc=False),
    )(body)


_sc_gather1 = _make_sc_gather(S1, _N1, _C1)
_sc_gather2 = _make_sc_gather(S2, _N2, _C2)


_BI = 1024


def _prep_body(w2_ref, r2_ref, g_ref, b8_ref):
    acc = w2_ref[:, 0:R] * r2_ref[:, 0:R]
    for k in range(1, K):
        acc = acc + w2_ref[:, k * R:(k + 1) * R] * r2_ref[:, k * R:(k + 1) * R]
    bt = lax.dot_general(
        g_ref[...], acc, (((1,), (1,)), ((), ())),
        preferred_element_type=jnp.float32)
    for k in range(K):
        b8_ref[k * R:(k + 1) * R, :] = bt


def _main_body(w1_ref, r1_ref, b8_ref, out_ref):
    p = w1_ref[...] * r1_ref[...]
    out_ref[...] = jnp.dot(p, b8_ref[...],
                           preferred_element_type=jnp.float32)


def kernel(x, M1, M2, G, idx1, idx2, dist1, dist2):
    del x
    r2 = _sc_gather2(M2, idx2.reshape(-1))
    r1 = _sc_gather1(M1, idx1.reshape(-1))
    r1f = r1.reshape(S1, K * R)
    r2f = r2.reshape(S2, K * R)
    we1 = jnp.repeat(dist1, R, axis=1)
    we2 = jnp.repeat(dist2, R, axis=1)

    b8 = pl.pallas_call(
        _prep_body,
        out_shape=jax.ShapeDtypeStruct((K * R, S2), jnp.float32),
    )(we2, r2f, G)

    out = pl.pallas_call(
        _main_body,
        grid=(S1 // _BI,),
        in_specs=[
            pl.BlockSpec((_BI, K * R), lambda i: (i, 0)),
            pl.BlockSpec((_BI, K * R), lambda i: (i, 0)),
            pl.BlockSpec((K * R, S2), lambda i: (0, 0)),
        ],
        out_specs=pl.BlockSpec((_BI, S2), lambda i: (i, 0)),
        out_shape=jax.ShapeDtypeStruct((S1, S2), jnp.float32),
        compiler_params=pltpu.CompilerParams(
            dimension_semantics=("parallel",)),
    )(we1, r1f, b8)
    return out


# speedup vs baseline: 5.1842x; 1.0279x over previous
import functools

import jax
import jax.numpy as jnp
from jax import lax
from jax.experimental import pallas as pl
from jax.experimental.pallas import tpu as pltpu
from jax.experimental.pallas import tpu_sc as plsc

S1, S2 = 8192, 4096
R = 16
K = 8

_info = plsc.get_sparse_core_info()
_NC, _NS = _info.num_cores, _info.num_subcores
_NW = _NC * _NS

_N1 = S1 * K
_N2 = S2 * K
_C1 = _N1 // _NW
_C2 = _N2 // _NW


def _sc_gather_body(m1_hbm, i1_hbm, m2_hbm, i2_hbm, o1_hbm, o2_hbm,
                    m1_sh, m2_sh, i1_v, r1_v, i2_v, r2_v, sem, sem2):
    sid = lax.axis_index("s")
    wid = sid * _NC + lax.axis_index("c")
    b1 = wid * _C1
    b2 = wid * _C2

    @pl.when(sid == 0)
    def _():
        pltpu.sync_copy(m2_hbm, m2_sh)

    @pl.when(sid == 1)
    def _():
        pltpu.sync_copy(m1_hbm, m1_sh)

    pltpu.sync_copy(i2_hbm.at[pl.ds(b2, _C2)], i2_v)
    pltpu.sync_copy(i1_hbm.at[pl.ds(b1, _C1)], i1_v)
    plsc.subcore_barrier()
    cp2 = pltpu.async_copy(m2_sh.at[i2_v], r2_v, sem2)
    cp1 = pltpu.async_copy(m1_sh.at[i1_v], r1_v, sem)
    cp2.wait()
    pltpu.sync_copy(r2_v, o2_hbm.at[pl.ds(b2, _C2)])
    cp1.wait()
    pltpu.sync_copy(r1_v, o1_hbm.at[pl.ds(b1, _C1)])


_sc_gather = functools.partial(
    pl.kernel,
    out_type=(
        jax.ShapeDtypeStruct((_N1, R), jnp.float32),
        jax.ShapeDtypeStruct((_N2, R), jnp.float32),
    ),
    mesh=plsc.VectorSubcoreMesh(core_axis_name="c", subcore_axis_name="s"),
    scratch_types=[
        pltpu.VMEM_SHARED((S1, R), jnp.float32),
        pltpu.VMEM_SHARED((S2, R), jnp.float32),
        pltpu.VMEM((_C1,), jnp.int32),
        pltpu.VMEM((_C1, R), jnp.float32),
        pltpu.VMEM((_C2,), jnp.int32),
        pltpu.VMEM((_C2, R), jnp.float32),
        pltpu.SemaphoreType.DMA,
        pltpu.SemaphoreType.DMA,
    ],
    compiler_params=pltpu.CompilerParams(use_tc_tiling_on_sc=False),
)(_sc_gather_body)


_BI = 1024


def _main_body(w2_ref, r2_ref, g_ref, w1_ref, r1_ref, out_ref, b8_ref):
    i = pl.program_id(0)

    @pl.when(i == 0)
    def _():
        acc = w2_ref[:, 0:R] * r2_ref[:, 0:R]
        for k in range(1, K):
            acc = acc + (w2_ref[:, k * R:(k + 1) * R]
                         * r2_ref[:, k * R:(k + 1) * R])
        bt = lax.dot_general(
            g_ref[...], acc, (((1,), (1,)), ((), ())),
            preferred_element_type=jnp.float32)
        for k in range(K):
            b8_ref[k * R:(k + 1) * R, :] = bt

    p = w1_ref[...] * r1_ref[...]
    out_ref[...] = jnp.dot(p, b8_ref[...],
                           preferred_element_type=jnp.float32)


def kernel(x, M1, M2, G, idx1, idx2, dist1, dist2):
    del x
    r1, r2 = _sc_gather(M1, idx1.reshape(-1), M2, idx2.reshape(-1))
    r1f = r1.reshape(S1, K * R)
    r2f = r2.reshape(S2, K * R)
    we1 = jnp.repeat(dist1, R, axis=1)
    we2 = jnp.repeat(dist2, R, axis=1)

    out = pl.pallas_call(
        _main_body,
        grid=(S1 // _BI,),
        in_specs=[
            pl.BlockSpec((S2, K * R), lambda i: (0, 0)),
            pl.BlockSpec((S2, K * R), lambda i: (0, 0)),
            pl.BlockSpec((R, R), lambda i: (0, 0)),
            pl.BlockSpec((_BI, K * R), lambda i: (i, 0)),
            pl.BlockSpec((_BI, K * R), lambda i: (i, 0)),
        ],
        out_specs=pl.BlockSpec((_BI, S2), lambda i: (i, 0)),
        out_shape=jax.ShapeDtypeStruct((S1, S2), jnp.float32),
        scratch_shapes=[pltpu.VMEM((K * R, S2), jnp.float32)],
        compiler_params=pltpu.CompilerParams(
            dimension_semantics=("arbitrary",)),
    )(we2, r2f, G, we1, r1f)
    return out
